# baseline (speedup 1.0000x reference)
import functools

import jax
import jax.numpy as jnp
from jax import lax
from jax.experimental import pallas as pl
from jax.experimental.pallas import tpu as pltpu


def _shift_lanes(y, s):
    if s == 0:
        return y
    c = y.shape[0]
    z = jnp.zeros((c, abs(s)), y.dtype)
    if s > 0:
        return jnp.concatenate([y[:, s:], z], axis=1)
    return jnp.concatenate([z, y[:, :s]], axis=1)


def _relu_bn(a, s, b):
    return jnp.maximum(a * s[...] + b[...], 0.0)


def _fused_kernel(
        xh0_ref, xh1_ref,
        pw_ref,
        b0w1, b0s1, b0b1, b0w2, b0s2, b0b2, b0w3, b0s3, b0b3, b0wd, b0sd, b0bd,
        b1w1, b1s1, b1b1, b1w2, b1s2, b1b2, b1w3, b1s3, b1b3,
        b2w1, b2s1, b2b1, b2w2, b2s2, b2b2, b2w3, b2s3, b2b3,
        o_ref,
        *, nb, ho, wo):
    ell = ho * wo
    half = 2 * ell
    big = nb * ell
    pos = lax.broadcasted_iota(jnp.int32, (1, big), 1)
    wq = pos % wo
    hq = (pos // wo) % ho
    zero = jnp.zeros((), jnp.bfloat16)

    def _mask(dy, dx):
        m = None
        for cond in ((hq >= -dy) if dy < 0 else (hq < ho - dy) if dy > 0 else None,
                     (wq >= -dx) if dx < 0 else (wq < wo - dx) if dx > 0 else None):
            if cond is not None:
                m = cond if m is None else m & cond
        return m

    masks = {(dy, dx): _mask(dy, dx)
             for dy in (-1, 0, 1) for dx in (-1, 0, 1) if (dy, dx) != (0, 0)}

    dimnum = (((0,), (0,)), ((), ()))
    std = (((1,), (0,)), ((), ()))

    def tconv(wmat, rhs):
        return lax.dot_general(wmat[...], rhs, dimnum,
                               preferred_element_type=jnp.float32)

    def conv3x3(w2, taps):
        acc = None
        for ky in range(3):
            for kx in range(3):
                y, dy, dx = taps(ky, kx)
                t = _shift_lanes(y, wo * dy + dx)
                if (dy, dx) != (0, 0):
                    t = jnp.where(masks[(dy, dx)], t, zero)
                d = lax.dot_general(w2[ky * 3 + kx], t, dimnum,
                                    preferred_element_type=jnp.float32)
                acc = d if acc is None else acc + d
        return acc

    def wide(ref):
        return jnp.concatenate([ref[k] for k in range(nb)], axis=1)

    pw = pw_ref[...]

    def wsplit(y):
        sel = [lax.dot_general(y[:, k * half:(k + 1) * half], pw, std,
                               preferred_element_type=jnp.float32
                               ).astype(jnp.bfloat16)
               for k in range(nb)]
        return (jnp.concatenate([s[:, :ell] for s in sel], axis=1),
                jnp.concatenate([s[:, ell:] for s in sel], axis=1))

    xh0 = wide(xh0_ref)
    y1p = {}
    for rp, xh in ((0, xh0), (1, wide(xh1_ref))):
        y1f = _relu_bn(tconv(b0w1, xh), b0s1, b0b1).astype(jnp.bfloat16)
        y1p[(rp, 0)], y1p[(rp, 1)] = wsplit(y1f)

    x_ee, _ = wsplit(xh0)

    def b0_taps(ky, kx):
        rp, dy = ((1, -1) if ky == 0 else (0, 0) if ky == 1 else (1, 0))
        cp, dx = ((1, -1) if kx == 0 else (0, 0) if kx == 1 else (1, 0))
        return y1p[(rp, cp)], dy, dx

    y2 = _relu_bn(conv3x3(b0w2, b0_taps), b0s2, b0b2).astype(jnp.bfloat16)

    a3 = tconv(b0w3, y2)
    ad = tconv(b0wd, x_ee)
    x_cur = jnp.maximum(a3 * b0s3[...] + b0b3[...]
                        + ad * b0sd[...] + b0bd[...], 0.0)

    for (w1, s1, bb1, w2, s2, bb2, w3, s3, bb3) in (
            (b1w1, b1s1, b1b1, b1w2, b1s2, b1b2, b1w3, b1s3, b1b3),
            (b2w1, b2s1, b2b1, b2w2, b2s2, b2b2, b2w3, b2s3, b2b3)):
        y1 = _relu_bn(tconv(w1, x_cur.astype(jnp.bfloat16)),
                      s1, bb1).astype(jnp.bfloat16)

        def b_taps(ky, kx, _y=y1):
            return _y, ky - 1, kx - 1

        y2 = _relu_bn(conv3x3(w2, b_taps), s2, bb2).astype(jnp.bfloat16)
        x_cur = jnp.maximum(tconv(w3, y2) * s3[...] + bb3[...] + x_cur, 0.0)

    for k in range(nb):
        o_ref[k] = x_cur[:, k * ell:(k + 1) * ell]


def _col(v):
    return v.reshape(v.shape[0], 1).astype(jnp.float32)


def _res_layer_forward(x, params):
    n, c, h, w = x.shape
    ho, wo = h // 2, w // 2
    ell = ho * wo
    half = ho * w

    b0, b1, b2 = params["blocks"]
    mid = b0["conv1"]["wmat"].shape[-1]
    cout = b0["conv3"]["wmat"].shape[-1]

    x5 = x.reshape(n, c, ho, 2, w)
    halves = [x5[:, :, :, rp, :].reshape(n, c, half).astype(jnp.bfloat16)
              for rp in (0, 1)]

    t = jnp.arange(half)
    tp, cp = t % ell, t // ell
    src = (tp // wo) * w + 2 * (tp % wo) + cp
    pw = (jnp.arange(half)[:, None] == src[None, :]).astype(jnp.bfloat16)

    def cbn(p):
        return [p["wmat"], _col(p["scale"]), _col(p["bias"])]

    args = halves + [pw]
    args += cbn(b0["conv1"]) + cbn(b0["conv2"]) + cbn(b0["conv3"]) + cbn(b0["down"])
    for blk in (b1, b2):
        args += cbn(blk["conv1"]) + cbn(blk["conv2"]) + cbn(blk["conv3"])

    grid = 1
    nb = n // grid

    def _batch_spec(shape):
        blk = (nb,) + tuple(shape[1:])
        return pl.BlockSpec(blk, lambda i: (i,) + (0,) * (len(shape) - 1))

    def _const_spec(shape):
        rank = len(shape)
        return pl.BlockSpec(tuple(shape), lambda i, _r=rank: (0,) * _r)

    in_specs = [_batch_spec(p.shape) for p in halves] \
             + [_const_spec(a.shape) for a in args[2:]]

    flops = 2 * ell * n * (4 * c * mid + 9 * mid * mid + mid * cout + c * cout)
    flops += 2 * 2 * ell * n * (cout * mid + 9 * mid * mid + mid * cout)
    flops += 2 * n * half * half * (2 * mid + c)
    bytes_accessed = int(sum(a.size * a.dtype.itemsize for a in args)) \
                   + n * cout * ell * 4

    out = pl.pallas_call(
        functools.partial(_fused_kernel, nb=nb, ho=ho, wo=wo),
        out_shape=jax.ShapeDtypeStruct((n, cout, ell), jnp.float32),
        grid_spec=pltpu.PrefetchScalarGridSpec(
            num_scalar_prefetch=0,
            grid=(grid,),
            in_specs=in_specs,
            out_specs=_batch_spec((n, cout, ell)),
        ),
        compiler_params=pltpu.CompilerParams(
            dimension_semantics=(pltpu.PARALLEL,)),
        cost_estimate=pl.CostEstimate(
            flops=int(flops), transcendentals=0, bytes_accessed=bytes_accessed),
    )(*args)
    return out.reshape(n, cout, ho, wo)


def kernel(x,
           b0_conv1_wmat, b0_conv1_w4d, b0_conv1_scale, b0_conv1_bias,
           b0_conv2_wmat, b0_conv2_w4d, b0_conv2_scale, b0_conv2_bias,
           b0_conv3_wmat, b0_conv3_w4d, b0_conv3_scale, b0_conv3_bias,
           b0_down_wmat, b0_down_w4d, b0_down_scale, b0_down_bias,
           b1_conv1_wmat, b1_conv1_w4d, b1_conv1_scale, b1_conv1_bias,
           b1_conv2_wmat, b1_conv2_w4d, b1_conv2_scale, b1_conv2_bias,
           b1_conv3_wmat, b1_conv3_w4d, b1_conv3_scale, b1_conv3_bias,
           b2_conv1_wmat, b2_conv1_w4d, b2_conv1_scale, b2_conv1_bias,
           b2_conv2_wmat, b2_conv2_w4d, b2_conv2_scale, b2_conv2_bias,
           b2_conv3_wmat, b2_conv3_w4d, b2_conv3_scale, b2_conv3_bias):
    def c(wmat, scale, bias):
        return {"wmat": wmat, "scale": scale, "bias": bias}
    params = {"blocks": [
        {"conv1": c(b0_conv1_wmat, b0_conv1_scale, b0_conv1_bias),
         "conv2": c(b0_conv2_wmat, b0_conv2_scale, b0_conv2_bias),
         "conv3": c(b0_conv3_wmat, b0_conv3_scale, b0_conv3_bias),
         "down": c(b0_down_wmat, b0_down_scale, b0_down_bias)},
        {"conv1": c(b1_conv1_wmat, b1_conv1_scale, b1_conv1_bias),
         "conv2": c(b1_conv2_wmat, b1_conv2_scale, b1_conv2_bias),
         "conv3": c(b1_conv3_wmat, b1_conv3_scale, b1_conv3_bias)},
        {"conv1": c(b2_conv1_wmat, b2_conv1_scale, b2_conv1_bias),
         "conv2": c(b2_conv2_wmat, b2_conv2_scale, b2_conv2_bias),
         "conv3": c(b2_conv3_wmat, b2_conv3_scale, b2_conv3_bias)},
    ]}
    return _res_layer_forward(x, params)


# speedup vs baseline: 1.0232x; 1.0232x over previous
import functools

import jax
import jax.numpy as jnp
from jax import lax
from jax.experimental import pallas as pl
from jax.experimental.pallas import tpu as pltpu


def _shift_lanes(y, s):
    if s == 0:
        return y
    c = y.shape[0]
    z = jnp.zeros((c, abs(s)), y.dtype)
    if s > 0:
        return jnp.concatenate([y[:, s:], z], axis=1)
    return jnp.concatenate([z, y[:, :s]], axis=1)


def _relu_bn(a, s, b):
    return jnp.maximum(a * s[...] + b[...], 0.0)


def _fused_kernel(
        xh0_ref, xh1_ref,
        pw_ref,
        b0w1, b0s1, b0b1, b0w2, b0s2, b0b2, b0w3, b0s3, b0b3, b0wd, b0sd, b0bd,
        b1w1, b1s1, b1b1, b1w2, b1s2, b1b2, b1w3, b1s3, b1b3,
        b2w1, b2s1, b2b1, b2w2, b2s2, b2b2, b2w3, b2s3, b2b3,
        o_ref,
        *, nb, ho, wo):
    ell = ho * wo
    half = 2 * ell
    big = nb * ell
    pos = lax.broadcasted_iota(jnp.int32, (1, big), 1)
    wq = pos % wo
    hq = (pos // wo) % ho
    zero = jnp.zeros((), jnp.bfloat16)

    def _mask(dy, dx):
        m = None
        for cond in ((hq >= -dy) if dy < 0 else (hq < ho - dy) if dy > 0 else None,
                     (wq >= -dx) if dx < 0 else (wq < wo - dx) if dx > 0 else None):
            if cond is not None:
                m = cond if m is None else m & cond
        return m

    masks = {(dy, dx): _mask(dy, dx)
             for dy in (-1, 0, 1) for dx in (-1, 0, 1) if (dy, dx) != (0, 0)}

    dimnum = (((0,), (0,)), ((), ()))
    std = (((1,), (0,)), ((), ()))

    def tconv(wmat, rhs):
        return lax.dot_general(wmat[...], rhs, dimnum,
                               preferred_element_type=jnp.float32)

    def conv3x3(w2, taps):
        acc = None
        for ky in range(3):
            for kx in range(3):
                y, dy, dx = taps(ky, kx)
                t = _shift_lanes(y, wo * dy + dx)
                if (dy, dx) != (0, 0):
                    t = jnp.where(masks[(dy, dx)], t, zero)
                d = lax.dot_general(w2[ky * 3 + kx], t, dimnum,
                                    preferred_element_type=jnp.float32)
                acc = d if acc is None else acc + d
        return acc

    def wide(ref):
        return jnp.concatenate([ref[k] for k in range(nb)], axis=1)

    pw = pw_ref[...]

    def wsplit(y):
        sel = [lax.dot_general(y[:, k * half:(k + 1) * half], pw, std,
                               preferred_element_type=jnp.float32
                               ).astype(jnp.bfloat16)
               for k in range(nb)]
        return (jnp.concatenate([s[:, :ell] for s in sel], axis=1),
                jnp.concatenate([s[:, ell:] for s in sel], axis=1))

    xh0 = wide(xh0_ref)
    y1p = {}
    for rp, xh in ((0, xh0), (1, wide(xh1_ref))):
        y1f = _relu_bn(tconv(b0w1, xh), b0s1, b0b1).astype(jnp.bfloat16)
        y1p[(rp, 0)], y1p[(rp, 1)] = wsplit(y1f)

    x_ee, _ = wsplit(xh0)

    def b0_taps(ky, kx):
        rp, dy = ((1, -1) if ky == 0 else (0, 0) if ky == 1 else (1, 0))
        cp, dx = ((1, -1) if kx == 0 else (0, 0) if kx == 1 else (1, 0))
        return y1p[(rp, cp)], dy, dx

    y2 = _relu_bn(conv3x3(b0w2, b0_taps), b0s2, b0b2).astype(jnp.bfloat16)

    a3 = tconv(b0w3, y2)
    ad = tconv(b0wd, x_ee)
    x_cur = jnp.maximum(a3 * b0s3[...] + b0b3[...]
                        + ad * b0sd[...] + b0bd[...], 0.0)

    for (w1, s1, bb1, w2, s2, bb2, w3, s3, bb3) in (
            (b1w1, b1s1, b1b1, b1w2, b1s2, b1b2, b1w3, b1s3, b1b3),
            (b2w1, b2s1, b2b1, b2w2, b2s2, b2b2, b2w3, b2s3, b2b3)):
        y1 = _relu_bn(tconv(w1, x_cur.astype(jnp.bfloat16)),
                      s1, bb1).astype(jnp.bfloat16)

        def b_taps(ky, kx, _y=y1):
            return _y, ky - 1, kx - 1

        y2 = _relu_bn(conv3x3(w2, b_taps), s2, bb2).astype(jnp.bfloat16)
        x_cur = jnp.maximum(tconv(w3, y2) * s3[...] + bb3[...] + x_cur, 0.0)

    for k in range(nb):
        o_ref[k] = x_cur[:, k * ell:(k + 1) * ell]


def _col(v):
    return v.reshape(v.shape[0], 1).astype(jnp.float32)


def _res_layer_forward(x, params):
    n, c, h, w = x.shape
    ho, wo = h // 2, w // 2
    ell = ho * wo
    half = ho * w

    b0, b1, b2 = params["blocks"]
    mid = b0["conv1"]["wmat"].shape[-1]
    cout = b0["conv3"]["wmat"].shape[-1]

    x5 = x.reshape(n, c, ho, 2, w)
    halves = [x5[:, :, :, rp, :].reshape(n, c, half).astype(jnp.bfloat16)
              for rp in (0, 1)]

    t = jnp.arange(half)
    tp, cp = t % ell, t // ell
    src = (tp // wo) * w + 2 * (tp % wo) + cp
    pw = (jnp.arange(half)[:, None] == src[None, :]).astype(jnp.bfloat16)

    def cbn(p):
        return [p["wmat"], _col(p["scale"]), _col(p["bias"])]

    args = halves + [pw]
    args += cbn(b0["conv1"]) + cbn(b0["conv2"]) + cbn(b0["conv3"]) + cbn(b0["down"])
    for blk in (b1, b2):
        args += cbn(blk["conv1"]) + cbn(blk["conv2"]) + cbn(blk["conv3"])

    grid = 2 if n % 2 == 0 else 1
    nb = n // grid

    def _batch_spec(shape):
        blk = (nb,) + tuple(shape[1:])
        return pl.BlockSpec(blk, lambda i: (i,) + (0,) * (len(shape) - 1))

    def _const_spec(shape):
        rank = len(shape)
        return pl.BlockSpec(tuple(shape), lambda i, _r=rank: (0,) * _r)

    in_specs = [_batch_spec(p.shape) for p in halves] \
             + [_const_spec(a.shape) for a in args[2:]]

    flops = 2 * ell * n * (4 * c * mid + 9 * mid * mid + mid * cout + c * cout)
    flops += 2 * 2 * ell * n * (cout * mid + 9 * mid * mid + mid * cout)
    flops += 2 * n * half * half * (2 * mid + c)
    bytes_accessed = int(sum(a.size * a.dtype.itemsize for a in args)) \
                   + n * cout * ell * 4

    out = pl.pallas_call(
        functools.partial(_fused_kernel, nb=nb, ho=ho, wo=wo),
        out_shape=jax.ShapeDtypeStruct((n, cout, ell), jnp.float32),
        grid_spec=pltpu.PrefetchScalarGridSpec(
            num_scalar_prefetch=0,
            grid=(grid,),
            in_specs=in_specs,
            out_specs=_batch_spec((n, cout, ell)),
        ),
        compiler_params=pltpu.CompilerParams(
            dimension_semantics=(pltpu.PARALLEL,)),
        cost_estimate=pl.CostEstimate(
            flops=int(flops), transcendentals=0, bytes_accessed=bytes_accessed),
    )(*args)
    return out.reshape(n, cout, ho, wo)


def kernel(x,
           b0_conv1_wmat, b0_conv1_w4d, b0_conv1_scale, b0_conv1_bias,
           b0_conv2_wmat, b0_conv2_w4d, b0_conv2_scale, b0_conv2_bias,
           b0_conv3_wmat, b0_conv3_w4d, b0_conv3_scale, b0_conv3_bias,
           b0_down_wmat, b0_down_w4d, b0_down_scale, b0_down_bias,
           b1_conv1_wmat, b1_conv1_w4d, b1_conv1_scale, b1_conv1_bias,
           b1_conv2_wmat, b1_conv2_w4d, b1_conv2_scale, b1_conv2_bias,
           b1_conv3_wmat, b1_conv3_w4d, b1_conv3_scale, b1_conv3_bias,
           b2_conv1_wmat, b2_conv1_w4d, b2_conv1_scale, b2_conv1_bias,
           b2_conv2_wmat, b2_conv2_w4d, b2_conv2_scale, b2_conv2_bias,
           b2_conv3_wmat, b2_conv3_w4d, b2_conv3_scale, b2_conv3_bias):
    def c(wmat, scale, bias):
        return {"wmat": wmat, "scale": scale, "bias": bias}
    params = {"blocks": [
        {"conv1": c(b0_conv1_wmat, b0_conv1_scale, b0_conv1_bias),
         "conv2": c(b0_conv2_wmat, b0_conv2_scale, b0_conv2_bias),
         "conv3": c(b0_conv3_wmat, b0_conv3_scale, b0_conv3_bias),
         "down": c(b0_down_wmat, b0_down_scale, b0_down_bias)},
        {"conv1": c(b1_conv1_wmat, b1_conv1_scale, b1_conv1_bias),
         "conv2": c(b1_conv2_wmat, b1_conv2_scale, b1_conv2_bias),
         "conv3": c(b1_conv3_wmat, b1_conv3_scale, b1_conv3_bias)},
        {"conv1": c(b2_conv1_wmat, b2_conv1_scale, b2_conv1_bias),
         "conv2": c(b2_conv2_wmat, b2_conv2_scale, b2_conv2_bias),
         "conv3": c(b2_conv3_wmat, b2_conv3_scale, b2_conv3_bias)},
    ]}
    return _res_layer_forward(x, params)
